# baseline (speedup 1.0000x reference)
import jax
import jax.numpy as jnp
from jax import lax
from jax.experimental import pallas as pl
from jax.experimental.pallas import tpu as pltpu
from jax.experimental.pallas import tpu_sc as plsc

NC, NS, L = 2, 16, 16
NW = NC * NS
IDX_CHUNK = 128


def _sc_edge_embedding(table, lidx, ridx):
    nw, k, _ = lidx.shape
    bpw = k * IDX_CHUNK
    d = table.shape[1]
    mesh = plsc.VectorSubcoreMesh(core_axis_name="c", subcore_axis_name="s")

    def body(table_hbm, lidx_hbm, ridx_hbm, out_hbm, liv, riv, lv, rv, ov, lsem, rsem, osem):
        wid = lax.axis_index("s") * NC + lax.axis_index("c")
        base = wid * bpw
        pltpu.sync_copy(lidx_hbm.at[wid], liv)
        pltpu.sync_copy(ridx_hbm.at[wid], riv)
        lcp, rcp, ocp = [], [], []
        for j in range(k):
            rows = pl.ds(j * IDX_CHUNK, IDX_CHUNK)
            lcp.append(pltpu.async_copy(table_hbm.at[liv.at[j]], lv.at[rows], lsem.at[j]))
            rcp.append(pltpu.async_copy(table_hbm.at[riv.at[j]], rv.at[rows], rsem.at[j]))
        for j in range(k):
            rows = pl.ds(j * IDX_CHUNK, IDX_CHUNK)
            lcp[j].wait()
            rcp[j].wait()

            @plsc.parallel_loop(j * IDX_CHUNK, (j + 1) * IDX_CHUNK, unroll=4)
            def mul_row(i):
                for c in range(d // L):
                    sl = pl.ds(c * L, L)
                    ov[i, 0, sl] = lv[i, sl] * rv[i, sl]

            ocp.append(pltpu.async_copy(
                ov.at[rows], out_hbm.at[pl.ds(base + j * IDX_CHUNK, IDX_CHUNK)], osem))
        for cp in ocp:
            cp.wait()

    run = pl.kernel(
        body,
        out_type=jax.ShapeDtypeStruct((nw * bpw, 1, d), jnp.float32),
        mesh=mesh,
        scratch_types=[
            pltpu.VMEM((k, IDX_CHUNK), jnp.int32),
            pltpu.VMEM((k, IDX_CHUNK), jnp.int32),
            pltpu.VMEM((bpw, d), jnp.float32),
            pltpu.VMEM((bpw, d), jnp.float32),
            pltpu.VMEM((bpw, 1, d), jnp.float32),
            pltpu.SemaphoreType.DMA((k,)),
            pltpu.SemaphoreType.DMA((k,)),
            pltpu.SemaphoreType.DMA,
        ],
        compiler_params=pltpu.CompilerParams(use_tc_tiling_on_sc=False),
    )
    return run(table, lidx, ridx)


def kernel(left_input, right_input, embedding):
    b = left_input.shape[0]
    lidx = left_input.reshape(NW, b // NW // IDX_CHUNK, IDX_CHUNK)
    ridx = right_input.reshape(NW, b // NW // IDX_CHUNK, IDX_CHUNK)
    return _sc_edge_embedding(embedding, lidx, ridx)


# speedup vs baseline: 1.0054x; 1.0054x over previous
import jax
import jax.numpy as jnp
from jax import lax
from jax.experimental import pallas as pl
from jax.experimental.pallas import tpu as pltpu
from jax.experimental.pallas import tpu_sc as plsc

NC, NS, L = 2, 16, 16
NW = NC * NS
IDX_CHUNK = 128


def _sc_edge_embedding(table, lidx, ridx):
    (b,) = lidx.shape
    d = table.shape[1]
    bpw = b // NW
    k = bpw // IDX_CHUNK
    mesh = plsc.VectorSubcoreMesh(core_axis_name="c", subcore_axis_name="s")

    def body(table_hbm, lidx_hbm, ridx_hbm, out_hbm, liv, riv, lv, rv, ov, lsem, rsem, osem):
        wid = lax.axis_index("s") * NC + lax.axis_index("c")
        base = wid * bpw
        pltpu.sync_copy(lidx_hbm.at[pl.ds(base, bpw)], liv)
        pltpu.sync_copy(ridx_hbm.at[pl.ds(base, bpw)], riv)
        lcp, rcp, ocp = [], [], []
        for j in range(k):
            rows = pl.ds(j * IDX_CHUNK, IDX_CHUNK)
            lcp.append(pltpu.async_copy(table_hbm.at[liv.at[rows]], lv.at[rows], lsem.at[j]))
            rcp.append(pltpu.async_copy(table_hbm.at[riv.at[rows]], rv.at[rows], rsem.at[j]))
        for j in range(k):
            rows = pl.ds(j * IDX_CHUNK, IDX_CHUNK)
            lcp[j].wait()
            rcp[j].wait()

            @plsc.parallel_loop(j * IDX_CHUNK, (j + 1) * IDX_CHUNK, unroll=4)
            def mul_row(i):
                for c in range(d // L):
                    sl = pl.ds(c * L, L)
                    ov[i, 0, sl] = lv[i, sl] * rv[i, sl]

            ocp.append(pltpu.async_copy(
                ov.at[rows], out_hbm.at[pl.ds(base + j * IDX_CHUNK, IDX_CHUNK)], osem))
        for cp in ocp:
            cp.wait()

    run = pl.kernel(
        body,
        out_type=jax.ShapeDtypeStruct((b, 1, d), jnp.float32),
        mesh=mesh,
        scratch_types=[
            pltpu.VMEM((bpw,), jnp.int32),
            pltpu.VMEM((bpw,), jnp.int32),
            pltpu.VMEM((bpw, d), jnp.float32),
            pltpu.VMEM((bpw, d), jnp.float32),
            pltpu.VMEM((bpw, 1, d), jnp.float32),
            pltpu.SemaphoreType.DMA((k,)),
            pltpu.SemaphoreType.DMA((k,)),
            pltpu.SemaphoreType.DMA,
        ],
        compiler_params=pltpu.CompilerParams(use_tc_tiling_on_sc=False),
    )
    return run(table, lidx, ridx)


def kernel(left_input, right_input, embedding):
    b = left_input.shape[0]
    return _sc_edge_embedding(embedding, left_input.reshape(b), right_input.reshape(b))


# speedup vs baseline: 1.0062x; 1.0007x over previous
import jax
import jax.numpy as jnp
from jax import lax
from jax.experimental import pallas as pl
from jax.experimental.pallas import tpu as pltpu
from jax.experimental.pallas import tpu_sc as plsc

NC, NS, L = 2, 16, 16
NW = NC * NS
IDX_CHUNK = 128


def _sc_edge_embedding(table, lidx, ridx):
    (b,) = lidx.shape
    v, d = table.shape
    bpw = b // NW
    k = bpw // IDX_CHUNK
    mesh = plsc.VectorSubcoreMesh(core_axis_name="c", subcore_axis_name="s")

    def body(table_hbm, lidx_hbm, ridx_hbm, out_hbm, liv, riv, lv, rv, ov,
             lsem, rsem, osem):
        wid = lax.axis_index("s") * NC + lax.axis_index("c")
        base = wid * bpw
        pltpu.sync_copy(lidx_hbm.at[pl.ds(base, bpw)], liv)
        pltpu.sync_copy(ridx_hbm.at[pl.ds(base, bpw)], riv)
        lcp, rcp, ocp = [], [], []
        for j in range(k):
            rows = pl.ds(j * IDX_CHUNK, IDX_CHUNK)
            lcp.append(pltpu.async_copy(table_hbm.at[liv.at[rows]], lv.at[rows], lsem.at[j]))
            rcp.append(pltpu.async_copy(table_hbm.at[riv.at[rows]], rv.at[rows], rsem.at[j]))
        for j in range(k):
            rows = pl.ds(j * IDX_CHUNK, IDX_CHUNK)
            lcp[j].wait()
            rcp[j].wait()

            @plsc.parallel_loop(j * IDX_CHUNK, (j + 1) * IDX_CHUNK, unroll=4)
            def mul_row(i):
                for c in range(d // L):
                    sl = pl.ds(c * L, L)
                    ov[i, 0, sl] = lv[i, sl] * rv[i, sl]

            ocp.append(pltpu.async_copy(
                ov.at[rows], out_hbm.at[pl.ds(base + j * IDX_CHUNK, IDX_CHUNK)], osem))
        for cp in ocp:
            cp.wait()

    run = pl.kernel(
        body,
        out_type=jax.ShapeDtypeStruct((b, 1, d), jnp.float32),
        mesh=mesh,
        scratch_types=[
            pltpu.VMEM((bpw,), jnp.int32),
            pltpu.VMEM((bpw,), jnp.int32),
            pltpu.VMEM((bpw, d), jnp.float32),
            pltpu.VMEM((bpw, d), jnp.float32),
            pltpu.VMEM((bpw, 1, d), jnp.float32),
            pltpu.SemaphoreType.DMA((k,)),
            pltpu.SemaphoreType.DMA((k,)),
            pltpu.SemaphoreType.DMA,
        ],
        compiler_params=pltpu.CompilerParams(use_tc_tiling_on_sc=False),
    )
    return run(table, lidx, ridx)


def kernel(left_input, right_input, embedding):
    b = left_input.shape[0]
    v, d = embedding.shape
    table = lax.optimization_barrier(embedding.reshape(v * d)).reshape(v, d)
    return _sc_edge_embedding(table, left_input.reshape(b), right_input.reshape(b))


# speedup vs baseline: 1.2672x; 1.2594x over previous
import jax
import jax.numpy as jnp
from jax import lax
from jax.experimental import pallas as pl
from jax.experimental.pallas import tpu as pltpu
from jax.experimental.pallas import tpu_sc as plsc

NC, NS, L = 2, 16, 16
NW = NC * NS
IDX_CHUNK = 128


def _sc_edge_embedding(table, lidx, ridx):
    nw, k, _ = lidx.shape
    bpw = k * IDX_CHUNK
    d = table.shape[1]
    mesh = plsc.VectorSubcoreMesh(core_axis_name="c", subcore_axis_name="s")

    def body(table_hbm, lidx_hbm, ridx_hbm, out_hbm, liv, riv, lv, rv, lsem, rsem, osem):
        wid = lax.axis_index("s") * NC + lax.axis_index("c")
        pltpu.sync_copy(lidx_hbm.at[wid], liv)
        pltpu.sync_copy(ridx_hbm.at[wid], riv)
        lcp, rcp, ocp = [], [], []
        for j in range(k):
            rows = pl.ds(j * IDX_CHUNK, IDX_CHUNK)
            lcp.append(pltpu.async_copy(table_hbm.at[liv.at[j]], lv.at[rows], lsem.at[j]))
            rcp.append(pltpu.async_copy(table_hbm.at[riv.at[j]], rv.at[rows], rsem.at[j]))
        for j in range(k):
            rows = pl.ds(j * IDX_CHUNK, IDX_CHUNK)
            lcp[j].wait()
            rcp[j].wait()

            @plsc.parallel_loop(j * IDX_CHUNK, (j + 1) * IDX_CHUNK, unroll=4)
            def mul_row(i):
                for c in range(d // L):
                    sl = pl.ds(c * L, L)
                    lv[i, sl] = lv[i, sl] * rv[i, sl]

            ocp.append(pltpu.async_copy(lv.at[rows], out_hbm.at[wid].at[rows], osem))
        for cp in ocp:
            cp.wait()

    run = pl.kernel(
        body,
        out_type=jax.ShapeDtypeStruct((nw, bpw, d), jnp.float32),
        mesh=mesh,
        scratch_types=[
            pltpu.VMEM((k, IDX_CHUNK), jnp.int32),
            pltpu.VMEM((k, IDX_CHUNK), jnp.int32),
            pltpu.VMEM((bpw, d), jnp.float32),
            pltpu.VMEM((bpw, d), jnp.float32),
            pltpu.SemaphoreType.DMA((k,)),
            pltpu.SemaphoreType.DMA((k,)),
            pltpu.SemaphoreType.DMA,
        ],
        compiler_params=pltpu.CompilerParams(use_tc_tiling_on_sc=False),
    )
    return run(table, lidx, ridx)


def kernel(left_input, right_input, embedding):
    b = left_input.shape[0]
    d = embedding.shape[1]
    lidx = left_input.reshape(NW, b // NW // IDX_CHUNK, IDX_CHUNK)
    ridx = right_input.reshape(NW, b // NW // IDX_CHUNK, IDX_CHUNK)
    out = _sc_edge_embedding(embedding, lidx, ridx)
    return out.reshape(b, 1, d)
